# Initial kernel scaffold; baseline (speedup 1.0000x reference)
import jax
import jax.numpy as jnp
from jax.experimental import pallas as pl


def kernel(embed_table, target, target_length):
    raise NotImplementedError("write your pallas kernel here")


# speedup vs baseline: 13.5569x; 13.5569x over previous
import functools

import jax
import jax.numpy as jnp
from jax import lax
from jax.experimental import pallas as pl
from jax.experimental.pallas import tpu as pltpu
from jax.experimental.pallas import tpu_sc as plsc

NC = 2
NS = 16
NW = NC * NS

BATCH = 4096
SEQ = 200
DIM = 64
VOCAB = 100000

NPT = BATCH // NW
CHUNK = 100
NCHUNK = SEQ // CHUNK
LANES = 16
NACC = DIM // LANES

_mesh = plsc.VectorSubcoreMesh(core_axis_name="c", subcore_axis_name="s")


@functools.partial(
    pl.kernel,
    out_type=jax.ShapeDtypeStruct((NW, NPT, DIM), jnp.float32),
    mesh=_mesh,
    compiler_params=pltpu.CompilerParams(use_tc_tiling_on_sc=False),
    scratch_types=[
        pltpu.VMEM((NPT, NCHUNK, CHUNK), jnp.int32),
        pltpu.VMEM((NCHUNK, CHUNK, DIM), jnp.float32),
        pltpu.VMEM((NCHUNK, CHUNK, DIM), jnp.float32),
        pltpu.VMEM((NPT + LANES,), jnp.int32),
        pltpu.VMEM((NPT, DIM), jnp.float32),
        pltpu.SemaphoreType.DMA,
        pltpu.SemaphoreType.DMA,
    ],
)
def _bow_pool(table_hbm, tgt_hbm, len_hbm, out_hbm,
              idx_v, rows0, rows1, len_v, out_v, sem0, sem1):
    wid = lax.axis_index("s") * NC + lax.axis_index("c")

    pltpu.sync_copy(tgt_hbm.at[wid], idx_v)
    pltpu.sync_copy(len_hbm.at[wid], len_v.at[pl.ds(0, NPT)])

    def gather(elem, rows_ref, sem, j):
        return pltpu.make_async_copy(
            table_hbm.at[idx_v.at[elem, j]], rows_ref.at[j], sem)

    def issue(elem, rows_ref, sem):
        for j in range(NCHUNK):
            gather(elem, rows_ref, sem, j).start()

    def wait(elem, rows_ref, sem):
        for j in range(NCHUNK):
            gather(elem, rows_ref, sem, j).wait()

    def compute(elem, rows_ref):
        def rbody(r, accs):
            out = []
            for c in range(NACC):
                a = accs[c]
                for j in range(NCHUNK):
                    a = a + rows_ref[j, r, pl.ds(c * LANES, LANES)]
                out.append(a)
            return tuple(out)

        zeros = tuple(jnp.zeros((LANES,), jnp.float32) for _ in range(NACC))
        accs = lax.fori_loop(0, CHUNK, rbody, zeros)
        lvv = len_v[pl.ds(elem, LANES)].astype(jnp.float32)
        inv = jnp.full((LANES,), 1.0, jnp.float32) / lvv
        scale = inv[0]
        for c in range(NACC):
            out_v[elem, pl.ds(c * LANES, LANES)] = accs[c] * scale

    issue(0, rows0, sem0)
    issue(1, rows1, sem1)

    def body(k, _):
        i = 2 * k
        wait(i, rows0, sem0)
        compute(i, rows0)
        issue(i + 2, rows0, sem0)
        wait(i + 1, rows1, sem1)
        compute(i + 1, rows1)
        issue(i + 3, rows1, sem1)
        return _

    lax.fori_loop(0, NPT // 2 - 1, body, None)

    wait(NPT - 2, rows0, sem0)
    compute(NPT - 2, rows0)
    wait(NPT - 1, rows1, sem1)
    compute(NPT - 1, rows1)

    pltpu.sync_copy(out_v, out_hbm.at[wid])


def kernel(embed_table, target, target_length):
    tgt = target.astype(jnp.int32).reshape(NW, NPT, NCHUNK, CHUNK)
    lens = target_length.astype(jnp.int32).reshape(NW, NPT)
    out = _bow_pool(embed_table.astype(jnp.float32), tgt, lens)
    return out.reshape(BATCH, DIM)


# speedup vs baseline: 13.6944x; 1.0101x over previous
import functools

import jax
import jax.numpy as jnp
from jax import lax
from jax.experimental import pallas as pl
from jax.experimental.pallas import tpu as pltpu
from jax.experimental.pallas import tpu_sc as plsc

NC = 2
NS = 16
NW = NC * NS

BATCH = 4096
SEQ = 200
DIM = 64

NPT = BATCH // NW
CHUNKS = (104, 96)
OFFS = (0, 104)
LANES = 16
NACC = DIM // LANES

_mesh = plsc.VectorSubcoreMesh(core_axis_name="c", subcore_axis_name="s")


@functools.partial(
    pl.kernel,
    out_type=jax.ShapeDtypeStruct((BATCH, DIM), jnp.float32),
    mesh=_mesh,
    compiler_params=pltpu.CompilerParams(use_tc_tiling_on_sc=False),
    scratch_types=[
        pltpu.VMEM((NPT, CHUNKS[0]), jnp.int32),
        pltpu.VMEM((NPT, CHUNKS[1]), jnp.int32),
        pltpu.VMEM((SEQ, DIM), jnp.float32),
        pltpu.VMEM((SEQ, DIM), jnp.float32),
        pltpu.VMEM((NPT + LANES,), jnp.int32),
        pltpu.VMEM((NPT, DIM), jnp.float32),
        pltpu.SemaphoreType.DMA,
        pltpu.SemaphoreType.DMA,
    ],
)
def _bow_pool(table_hbm, tgt_hbm, len_hbm, out_hbm,
              idx0, idx1, rows0, rows1, len_v, out_v, sem0, sem1):
    wid = lax.axis_index("s") * NC + lax.axis_index("c")
    base = wid * NPT
    idx = (idx0, idx1)

    for j in range(2):
        pltpu.sync_copy(
            tgt_hbm.at[pl.ds(base, NPT), pl.ds(OFFS[j], CHUNKS[j])],
            idx[j])
    pltpu.sync_copy(len_hbm.at[pl.ds(base, NPT)], len_v.at[pl.ds(0, NPT)])

    def gather(elem, rows_ref, sem, j):
        return pltpu.make_async_copy(
            table_hbm.at[idx[j].at[elem]],
            rows_ref.at[pl.ds(OFFS[j], CHUNKS[j])], sem)

    def issue(elem, rows_ref, sem):
        for j in range(2):
            gather(elem, rows_ref, sem, j).start()

    def wait(elem, rows_ref, sem):
        for j in range(2):
            gather(elem, rows_ref, sem, j).wait()

    def compute(elem, rows_ref):
        def rbody(r, accs):
            return tuple(
                accs[c] + rows_ref[r, pl.ds(c * LANES, LANES)]
                for c in range(NACC))

        zeros = tuple(jnp.zeros((LANES,), jnp.float32) for _ in range(NACC))
        accs = lax.fori_loop(0, SEQ, rbody, zeros)
        lvv = len_v[pl.ds(elem, LANES)].astype(jnp.float32)
        inv = jnp.full((LANES,), 1.0, jnp.float32) / lvv
        scale = inv[0]
        for c in range(NACC):
            out_v[elem, pl.ds(c * LANES, LANES)] = accs[c] * scale

    issue(0, rows0, sem0)
    issue(1, rows1, sem1)

    def body(k, _):
        i = 2 * k
        wait(i, rows0, sem0)
        compute(i, rows0)
        issue(i + 2, rows0, sem0)
        wait(i + 1, rows1, sem1)
        compute(i + 1, rows1)
        issue(i + 3, rows1, sem1)
        return _

    lax.fori_loop(0, NPT // 2 - 1, body, None)

    wait(NPT - 2, rows0, sem0)
    compute(NPT - 2, rows0)
    wait(NPT - 1, rows1, sem1)
    compute(NPT - 1, rows1)

    pltpu.sync_copy(out_v, out_hbm.at[pl.ds(base, NPT)])


def kernel(embed_table, target, target_length):
    return _bow_pool(embed_table,
                     target.astype(jnp.int32),
                     target_length.astype(jnp.int32))
